# Initial kernel scaffold; baseline (speedup 1.0000x reference)
import functools

import jax
import jax.numpy as jnp
from jax import lax
from jax.experimental import pallas as pl
from jax.experimental.pallas import tpu as pltpu
from jax.experimental.pallas import tpu_sc as plsc

ALPHA = 0.2
EPS = 9e-15
D = 128
DE = 144
CH = 128
LANES = 16
NCORES = 2
NSUB = 16


def _prep_body(x_ref, w_ref, attn_ref, hext_ref, st_ref):
    h = jnp.dot(x_ref[...], w_ref[...],
                preferred_element_type=jnp.float32,
                precision=lax.Precision.HIGHEST)
    blk = h.shape[0]
    hext_ref[...] = jnp.concatenate(
        [h, jnp.ones((blk, 1), jnp.float32), jnp.zeros((blk, DE - D - 1), jnp.float32)],
        axis=1)
    a = attn_ref[...].reshape(2, D)
    st_ref[...] = lax.dot_general(h, a, (((1,), (1,)), ((), ())),
                                  precision=lax.Precision.HIGHEST,
                                  preferred_element_type=jnp.float32)


def _combine_body(p_ref, o_ref):
    p = p_ref[0] + p_ref[1]
    o_ref[...] = p[:, :D] / (p[:, D:D + 1] + EPS)


def _make_sc_kernel(n, e):
    nchunk = e // CH
    chunks_per_core = nchunk // NCORES
    rows_per_sub = n // NSUB
    zcopy = 125
    nzcopy = rows_per_sub // zcopy

    mesh = plsc.VectorSubcoreMesh(core_axis_name="c", subcore_axis_name="s")

    @functools.partial(
        pl.kernel,
        out_type=jax.ShapeDtypeStruct((NCORES, n, DE), jnp.float32),
        mesh=mesh,
        scratch_types=[
            pltpu.VMEM((n, 2), jnp.float32),
            pltpu.VMEM((CH,), jnp.int32),
            pltpu.VMEM((CH,), jnp.int32),
            pltpu.VMEM((CH, DE), jnp.float32),
            pltpu.VMEM((CH,), jnp.float32),
            pltpu.VMEM_SHARED((n, DE), jnp.float32),
            pltpu.SemaphoreType.DMA,
        ],
    )
    def sc_kernel(hext_hbm, st_hbm, e0_hbm, e1_hbm, out_hbm,
                  st_v, e0_v, e1_v, rows_v, w_v, acc_sh, sem):
        core = lax.axis_index("c")
        sid = lax.axis_index("s")

        pltpu.sync_copy(st_hbm, st_v)

        zz = jnp.zeros((LANES,), jnp.float32)

        @pl.loop(0, CH)
        def _(j):
            for k in range(DE // LANES):
                rows_v[j, pl.ds(k * LANES, LANES)] = zz

        zbase = sid * rows_per_sub
        for r in range(nzcopy):
            pltpu.sync_copy(rows_v.at[pl.ds(0, zcopy)],
                            acc_sh.at[pl.ds(zbase + r * zcopy, zcopy)])
        plsc.subcore_barrier()

        z16 = jnp.zeros((LANES,), jnp.int32)
        o16 = jnp.ones((LANES,), jnp.int32)
        lo = core * chunks_per_core + sid
        hi = (core + 1) * chunks_per_core

        @pl.loop(lo, hi, step=NSUB)
        def _(ci):
            eb = ci * CH
            pltpu.sync_copy(e0_hbm.at[pl.ds(eb, CH)], e0_v)
            pltpu.sync_copy(e1_hbm.at[pl.ds(eb, CH)], e1_v)
            pltpu.async_copy(hext_hbm.at[e1_v], rows_v, sem).wait()

            for g in range(CH // LANES):
                i0 = e0_v[pl.ds(g * LANES, LANES)]
                i1 = e1_v[pl.ds(g * LANES, LANES)]
                sv = plsc.load_gather(st_v, [i0, z16])
                tv = plsc.load_gather(st_v, [i1, o16])
                v = sv + tv
                v = jnp.where(v > 0, v, ALPHA * v)
                w_v[pl.ds(g * LANES, LANES)] = jnp.exp(v)

            @pl.loop(0, CH)
            def _(j):
                wb = jnp.full((LANES,), w_v[j], jnp.float32)
                for k in range(DE // LANES):
                    sl = pl.ds(k * LANES, LANES)
                    rows_v[j, sl] = rows_v[j, sl] * wb

            pltpu.sync_copy(rows_v, acc_sh.at[e0_v], add=True)

        plsc.subcore_barrier()

        for r in range(nzcopy):
            sl = pl.ds(zbase + r * zcopy, zcopy)
            pltpu.sync_copy(acc_sh.at[sl], out_hbm.at[core].at[sl])

    return sc_kernel


def kernel(x, edge_index, W, attn):
    n = x.shape[0]
    e = edge_index.shape[1]
    nblk = 10
    blk = n // nblk

    hext, st = pl.pallas_call(
        _prep_body,
        grid=(nblk,),
        in_specs=[
            pl.BlockSpec((blk, D), lambda i: (i, 0)),
            pl.BlockSpec((D, D), lambda i: (0, 0)),
            pl.BlockSpec((1, 2 * D), lambda i: (0, 0)),
        ],
        out_specs=[
            pl.BlockSpec((blk, DE), lambda i: (i, 0)),
            pl.BlockSpec((blk, 2), lambda i: (i, 0)),
        ],
        out_shape=[
            jax.ShapeDtypeStruct((n, DE), jnp.float32),
            jax.ShapeDtypeStruct((n, 2), jnp.float32),
        ],
    )(x, W, attn)

    partials = _make_sc_kernel(n, e)(hext, st, edge_index[0], edge_index[1])

    out = pl.pallas_call(
        _combine_body,
        grid=(nblk,),
        in_specs=[pl.BlockSpec((NCORES, blk, DE), lambda i: (0, i, 0))],
        out_specs=pl.BlockSpec((blk, D), lambda i: (i, 0)),
        out_shape=jax.ShapeDtypeStruct((n, D), jnp.float32),
    )(partials)
    return out


# speedup vs baseline: 8.2962x; 8.2962x over previous
import dataclasses
import functools

import jax
import jax.numpy as jnp
from jax import lax
from jax.experimental import pallas as pl
from jax.experimental.pallas import tpu as pltpu
from jax.experimental.pallas import tpu_sc as plsc

ALPHA = 0.2
EPS = 9e-15
D = 128
CH = 128
LANES = 16
NCORES = 2
NSUB = 16
NPAD = 10240


def _prep_body(x_ref, w_ref, attn_ref, h_ref, s_ref, t_ref):
    h = jnp.dot(x_ref[...], w_ref[...],
                preferred_element_type=jnp.float32,
                precision=lax.Precision.HIGHEST)
    h_ref[...] = h
    a = attn_ref[...].reshape(2, D)
    st = lax.dot_general(h, a, (((1,), (1,)), ((), ())),
                         precision=lax.Precision.HIGHEST,
                         preferred_element_type=jnp.float32)
    s_ref[...] = st[:, 0]
    t_ref[...] = st[:, 1]


def _combine_body(p_ref, r_ref, o_ref):
    n = o_ref.shape[0]
    p = p_ref[0] + p_ref[1]
    rs = r_ref[0, :n] + r_ref[1, :n]
    o_ref[...] = p / (rs[:, None] + EPS)


def _make_sc_kernel(n, e):
    nchunk = e // CH
    chunks_per_core = nchunk // NCORES
    zrows = 80
    sub_elems = NPAD // NSUB

    mesh = plsc.VectorSubcoreMesh(core_axis_name="c", subcore_axis_name="s")
    cp = pltpu.CompilerParams()
    if "needs_layout_passes" in pltpu.CompilerParams.__dataclass_fields__:
        cp = dataclasses.replace(cp, needs_layout_passes=False)

    @functools.partial(
        pl.kernel,
        out_type=[
            jax.ShapeDtypeStruct((NCORES, n, D), jnp.float32),
            jax.ShapeDtypeStruct((NCORES, NPAD), jnp.float32),
        ],
        mesh=mesh,
        compiler_params=cp,
        scratch_types=[
            pltpu.VMEM((n,), jnp.float32),
            pltpu.VMEM((n,), jnp.float32),
            pltpu.VMEM((CH,), jnp.int32),
            pltpu.VMEM((CH,), jnp.int32),
            pltpu.VMEM((CH, D), jnp.float32),
            pltpu.VMEM((CH,), jnp.float32),
            pltpu.VMEM((sub_elems,), jnp.float32),
            pltpu.VMEM_SHARED((NPAD, D), jnp.float32),
            pltpu.VMEM_SHARED((NPAD,), jnp.float32),
            pltpu.SemaphoreType.DMA,
        ],
    )
    def sc_kernel(h_hbm, s_hbm, t_hbm, e0_hbm, e1_hbm, out_hbm, rs_hbm,
                  s_v, t_v, e0_v, e1_v, rows_v, w_v, z1_v, acc_sh, acc1_sh, sem):
        core = lax.axis_index("c")
        sid = lax.axis_index("s")

        pltpu.sync_copy(s_hbm, s_v)
        pltpu.sync_copy(t_hbm, t_v)

        zz = jnp.zeros((LANES,), jnp.float32)

        @pl.loop(0, CH)
        def _(j):
            for k in range(D // LANES):
                rows_v[j, pl.ds(k * LANES, LANES)] = zz

        @pl.loop(0, sub_elems // LANES)
        def _(j):
            z1_v[pl.ds(j * LANES, LANES)] = zz

        @pl.loop(sid, NPAD // zrows, step=NSUB)
        def _(t):
            off = pl.multiple_of(t * zrows, 8)
            pltpu.sync_copy(rows_v.at[pl.ds(0, zrows)],
                            acc_sh.at[pl.ds(off, zrows)])

        off1 = pl.multiple_of(sid * sub_elems, 128)
        pltpu.sync_copy(z1_v, acc1_sh.at[pl.ds(off1, sub_elems)])
        plsc.subcore_barrier()

        lo = core * chunks_per_core + sid
        hi = (core + 1) * chunks_per_core

        @pl.loop(lo, hi, step=NSUB)
        def _(ci):
            eb = pl.multiple_of(ci * CH, 8)
            pltpu.sync_copy(e0_hbm.at[pl.ds(eb, CH)], e0_v)
            pltpu.sync_copy(e1_hbm.at[pl.ds(eb, CH)], e1_v)
            pltpu.async_copy(h_hbm.at[e1_v], rows_v, sem).wait()

            @pl.loop(0, CH // LANES)
            def _(g):
                i0 = e0_v[pl.ds(g * LANES, LANES)]
                i1 = e1_v[pl.ds(g * LANES, LANES)]
                sv = plsc.load_gather(s_v, [i0])
                tv = plsc.load_gather(t_v, [i1])
                v = sv + tv
                v = jnp.where(v > 0, v, ALPHA * v)
                w_v[pl.ds(g * LANES, LANES)] = jnp.exp(v)
                for j in range(LANES):
                    wb = plsc.load_gather(
                        w_v, [jnp.full((LANES,), g * LANES + j, jnp.int32)])
                    row = g * LANES + j
                    for k in range(D // LANES):
                        sl = pl.ds(k * LANES, LANES)
                        rows_v[row, sl] = rows_v[row, sl] * wb

            pltpu.sync_copy(rows_v, acc_sh.at[e0_v], add=True)
            pltpu.sync_copy(w_v, acc1_sh.at[e0_v], add=True)

        plsc.subcore_barrier()

        @pl.loop(sid, n // zrows, step=NSUB)
        def _(t):
            sl = pl.ds(pl.multiple_of(t * zrows, 8), zrows)
            pltpu.sync_copy(acc_sh.at[sl], out_hbm.at[core].at[sl])

        sl1 = pl.ds(off1, sub_elems)
        pltpu.sync_copy(acc1_sh.at[sl1], rs_hbm.at[core].at[sl1])

    return sc_kernel


def kernel(x, edge_index, W, attn):
    n = x.shape[0]
    e = edge_index.shape[1]
    nblk = 10
    blk = n // nblk

    h, s, t = pl.pallas_call(
        _prep_body,
        out_shape=[
            jax.ShapeDtypeStruct((n, D), jnp.float32),
            jax.ShapeDtypeStruct((n,), jnp.float32),
            jax.ShapeDtypeStruct((n,), jnp.float32),
        ],
    )(x, W, attn)

    partials, rowsums = _make_sc_kernel(n, e)(h, s, t, edge_index[0], edge_index[1])

    out = pl.pallas_call(
        _combine_body,
        out_shape=jax.ShapeDtypeStruct((n, D), jnp.float32),
    )(partials, rowsums)
    return out


# speedup vs baseline: 14.1772x; 1.7089x over previous
import dataclasses
import functools

import jax
import jax.numpy as jnp
from jax import lax
from jax.experimental import pallas as pl
from jax.experimental.pallas import tpu as pltpu
from jax.experimental.pallas import tpu_sc as plsc

ALPHA = 0.2
EPS = 9e-15
D = 128
CH = 80
LANES = 16
NCORES = 2
NSUB = 16
NPAD = 10240


def _prep_body(x_ref, w_ref, attn_ref, h_ref, s_ref, t_ref):
    h = jnp.dot(x_ref[...], w_ref[...],
                preferred_element_type=jnp.float32,
                precision=lax.Precision.HIGHEST)
    h_ref[...] = h
    a = attn_ref[...].reshape(2, D)
    st = lax.dot_general(h, a, (((1,), (1,)), ((), ())),
                         precision=lax.Precision.HIGHEST,
                         preferred_element_type=jnp.float32)
    s_ref[...] = st[:, 0]
    t_ref[...] = st[:, 1]


def _combine_body(p_ref, r_ref, o_ref):
    n = o_ref.shape[0]
    p = p_ref[0] + p_ref[1]
    rs = r_ref[0, :n] + r_ref[1, :n]
    o_ref[...] = p / (rs[:, None] + EPS)


def _sc_compiler_params():
    cp = pltpu.CompilerParams()
    if "needs_layout_passes" in pltpu.CompilerParams.__dataclass_fields__:
        cp = dataclasses.replace(cp, needs_layout_passes=False)
    return cp


def _make_scA(n, e):
    nchunk = e // CH
    chunks_per_core = nchunk // NCORES
    csub = chunks_per_core // NSUB
    NBUF = 4

    mesh = plsc.VectorSubcoreMesh(core_axis_name="c", subcore_axis_name="s")

    scratch = [
        pltpu.VMEM((n,), jnp.float32),
        pltpu.VMEM((n,), jnp.float32),
    ]
    scratch += [pltpu.VMEM((CH,), jnp.int32) for _ in range(2 * NBUF)]
    scratch += [pltpu.VMEM((CH,), jnp.float32) for _ in range(NBUF)]
    scratch += [pltpu.SemaphoreType.DMA for _ in range(2 * NBUF)]

    @functools.partial(
        pl.kernel,
        out_type=jax.ShapeDtypeStruct((e,), jnp.float32),
        mesh=mesh,
        compiler_params=_sc_compiler_params(),
        scratch_types=scratch,
    )
    def scA(s_hbm, t_hbm, e0_hbm, e1_hbm, w_hbm, s_v, t_v, *bufs):
        e0b = bufs[0:NBUF]
        e1b = bufs[NBUF:2 * NBUF]
        wvb = bufs[2 * NBUF:3 * NBUF]
        isem = bufs[3 * NBUF:4 * NBUF]
        wsem = bufs[4 * NBUF:5 * NBUF]

        core = lax.axis_index("c")
        sid = lax.axis_index("s")

        pltpu.sync_copy(s_hbm, s_v)
        pltpu.sync_copy(t_hbm, t_v)

        def chunk_off(c):
            ci = core * chunks_per_core + sid + NSUB * c
            return pl.multiple_of(ci * CH, 8)

        def start_idx(c, b):
            eb = chunk_off(c)
            pltpu.make_async_copy(e0_hbm.at[pl.ds(eb, CH)], e0b[b], isem[b]).start()
            pltpu.make_async_copy(e1_hbm.at[pl.ds(eb, CH)], e1b[b], isem[b]).start()

        def wait_idx(b):
            pltpu.make_async_copy(e0_hbm.at[pl.ds(0, CH)], e0b[b], isem[b]).wait()
            pltpu.make_async_copy(e1_hbm.at[pl.ds(0, CH)], e1b[b], isem[b]).wait()

        def compute_w(c, b):
            @pl.loop(0, CH // LANES)
            def _(g):
                i0 = e0b[b][pl.ds(g * LANES, LANES)]
                i1 = e1b[b][pl.ds(g * LANES, LANES)]
                v = plsc.load_gather(s_v, [i0]) + plsc.load_gather(t_v, [i1])
                v = jnp.where(v > 0, v, ALPHA * v)
                wvb[b][pl.ds(g * LANES, LANES)] = jnp.exp(v)
            eb = chunk_off(c)
            pltpu.make_async_copy(wvb[b], w_hbm.at[pl.ds(eb, CH)], wsem[b]).start()

        def wait_wb(b):
            pltpu.make_async_copy(wvb[b], w_hbm.at[pl.ds(0, CH)], wsem[b]).wait()

        def iter_(c, b, b1, *, w_wb, do_idx, do_compute):
            if w_wb:
                wait_wb(b)
            if do_idx:
                start_idx(c, b)
            if do_compute:
                wait_idx(b1)
                compute_w(c - 1, b1)

        for c in range(4):
            iter_(c, c % NBUF, (c - 1) % NBUF,
                  w_wb=False, do_idx=True, do_compute=(c >= 1))

        @pl.loop(0, (124 - 4) // NBUF)
        def _(m):
            for slot in range(NBUF):
                c = 4 + m * NBUF + slot
                iter_(c, slot, (slot - 1) % NBUF,
                      w_wb=True, do_idx=True, do_compute=True)

        for c in range(124, csub + 1):
            iter_(c, c % NBUF, (c - 1) % NBUF,
                  w_wb=True, do_idx=(c <= csub - 1), do_compute=True)

        for cc in range(csub - 3, csub):
            wait_wb(cc % NBUF)

    return scA


def _make_scB(n, e):
    nchunk = e // CH
    chunks_per_core = nchunk // NCORES
    csub = chunks_per_core // NSUB
    zrows = 80
    sub_elems = NPAD // NSUB
    NBUF = 4

    mesh = plsc.VectorSubcoreMesh(core_axis_name="c", subcore_axis_name="s")

    scratch = [
        pltpu.VMEM((sub_elems,), jnp.float32),
        pltpu.VMEM_SHARED((NPAD, D), jnp.float32),
        pltpu.VMEM_SHARED((NPAD,), jnp.float32),
    ]
    scratch += [pltpu.VMEM((CH,), jnp.int32) for _ in range(2 * NBUF)]
    scratch += [pltpu.VMEM((CH, D), jnp.float32) for _ in range(NBUF)]
    scratch += [pltpu.VMEM((CH,), jnp.float32) for _ in range(NBUF)]
    scratch += [pltpu.SemaphoreType.DMA for _ in range(3 * NBUF)]

    @functools.partial(
        pl.kernel,
        out_type=[
            jax.ShapeDtypeStruct((NCORES, n, D), jnp.float32),
            jax.ShapeDtypeStruct((NCORES, NPAD), jnp.float32),
        ],
        mesh=mesh,
        compiler_params=_sc_compiler_params(),
        scratch_types=scratch,
    )
    def scB(h_hbm, w_hbm, e0_hbm, e1_hbm, out_hbm, rs_hbm,
            z1_v, acc_sh, acc1_sh, *bufs):
        e0b = bufs[0:NBUF]
        e1b = bufs[NBUF:2 * NBUF]
        rowsb = bufs[2 * NBUF:3 * NBUF]
        wvb = bufs[3 * NBUF:4 * NBUF]
        isem = bufs[4 * NBUF:5 * NBUF]
        gsem = bufs[5 * NBUF:6 * NBUF]
        ssem = bufs[6 * NBUF:7 * NBUF]

        core = lax.axis_index("c")
        sid = lax.axis_index("s")

        zz = jnp.zeros((LANES,), jnp.float32)

        @pl.loop(0, CH)
        def _(j):
            for k in range(D // LANES):
                rowsb[0][j, pl.ds(k * LANES, LANES)] = zz

        @pl.loop(0, sub_elems // LANES)
        def _(j):
            z1_v[pl.ds(j * LANES, LANES)] = zz

        @pl.loop(sid, NPAD // zrows, step=NSUB)
        def _(t):
            off = pl.multiple_of(t * zrows, 8)
            pltpu.sync_copy(rowsb[0].at[pl.ds(0, zrows)],
                            acc_sh.at[pl.ds(off, zrows)])

        off1 = pl.multiple_of(sid * sub_elems, 128)
        pltpu.sync_copy(z1_v, acc1_sh.at[pl.ds(off1, sub_elems)])
        plsc.subcore_barrier()

        def start_inputs(c, b):
            ci = core * chunks_per_core + sid + NSUB * c
            eb = pl.multiple_of(ci * CH, 8)
            pltpu.make_async_copy(e0_hbm.at[pl.ds(eb, CH)], e0b[b], isem[b]).start()
            pltpu.make_async_copy(e1_hbm.at[pl.ds(eb, CH)], e1b[b], isem[b]).start()
            pltpu.make_async_copy(w_hbm.at[pl.ds(eb, CH)], wvb[b], isem[b]).start()

        def wait_inputs(b):
            pltpu.make_async_copy(e0_hbm.at[pl.ds(0, CH)], e0b[b], isem[b]).wait()
            pltpu.make_async_copy(e1_hbm.at[pl.ds(0, CH)], e1b[b], isem[b]).wait()
            pltpu.make_async_copy(w_hbm.at[pl.ds(0, CH)], wvb[b], isem[b]).wait()

        def start_gather(b):
            pltpu.make_async_copy(h_hbm.at[e1b[b]], rowsb[b], gsem[b]).start()

        def wait_gather(b):
            pltpu.make_async_copy(h_hbm.at[e1b[b]], rowsb[b], gsem[b]).wait()

        def scale(b):
            @pl.loop(0, CH // LANES)
            def _(g):
                @pl.loop(0, LANES, step=4)
                def _(j):
                    for jj in range(4):
                        wbr = plsc.load_gather(
                            wvb[b],
                            [g * LANES + j + jj + jnp.zeros((LANES,), jnp.int32)])
                        row = g * LANES + j + jj
                        for k in range(D // LANES):
                            sl = pl.ds(k * LANES, LANES)
                            rowsb[b][row, sl] = rowsb[b][row, sl] * wbr

        def start_scatter(b):
            pltpu.make_async_copy(rowsb[b], acc_sh.at[e0b[b]], ssem[b]).start(add=True)
            pltpu.make_async_copy(wvb[b], acc1_sh.at[e0b[b]], ssem[b]).start(add=True)

        def wait_scatter(b):
            pltpu.make_async_copy(rowsb[b], acc_sh.at[e0b[b]], ssem[b]).wait()
            pltpu.make_async_copy(wvb[b], acc1_sh.at[e0b[b]], ssem[b]).wait()

        def iter_(c, b, b1, b2, *, w_scatter, do_idx, do_gather, do_compute):
            if w_scatter:
                wait_scatter(b)
            if do_idx:
                start_inputs(c, b)
            if do_gather:
                wait_inputs(b1)
                start_gather(b1)
            if do_compute:
                wait_gather(b2)
                scale(b2)
                start_scatter(b2)

        for c in range(4):
            iter_(c, c % NBUF, (c - 1) % NBUF, (c - 2) % NBUF,
                  w_scatter=False, do_idx=True, do_gather=(c >= 1),
                  do_compute=(c >= 2))

        @pl.loop(0, (120 - 4) // NBUF)
        def _(m):
            for slot in range(NBUF):
                c = 4 + m * NBUF + slot
                iter_(c, slot, (slot - 1) % NBUF, (slot - 2) % NBUF,
                      w_scatter=True, do_idx=True, do_gather=True,
                      do_compute=True)

        for c in range(120, csub + 2):
            iter_(c, c % NBUF, (c - 1) % NBUF, (c - 2) % NBUF,
                  w_scatter=(c - NBUF <= csub - 1), do_idx=(c <= csub - 1),
                  do_gather=(c - 1 <= csub - 1), do_compute=True)

        wait_scatter((csub - 2) % NBUF)
        wait_scatter((csub - 1) % NBUF)

        plsc.subcore_barrier()

        @pl.loop(sid, n // zrows, step=NSUB)
        def _(t):
            sl = pl.ds(pl.multiple_of(t * zrows, 8), zrows)
            pltpu.sync_copy(acc_sh.at[sl], out_hbm.at[core].at[sl])

        sl1 = pl.ds(off1, sub_elems)
        pltpu.sync_copy(acc1_sh.at[sl1], rs_hbm.at[core].at[sl1])

    return scB


def kernel(x, edge_index, W, attn):
    n = x.shape[0]
    e = edge_index.shape[1]

    h, s, t = pl.pallas_call(
        _prep_body,
        out_shape=[
            jax.ShapeDtypeStruct((n, D), jnp.float32),
            jax.ShapeDtypeStruct((n,), jnp.float32),
            jax.ShapeDtypeStruct((n,), jnp.float32),
        ],
    )(x, W, attn)

    e0 = edge_index[0]
    e1 = edge_index[1]
    w_edge = _make_scA(n, e)(s, t, e0, e1)
    partials, rowsums = _make_scB(n, e)(h, w_edge, e0, e1)

    out = pl.pallas_call(
        _combine_body,
        out_shape=jax.ShapeDtypeStruct((n, D), jnp.float32),
    )(partials, rowsums)
    return out


# speedup vs baseline: 18.6415x; 1.3149x over previous
import dataclasses
import functools

import jax
import jax.numpy as jnp
from jax import lax
from jax.experimental import pallas as pl
from jax.experimental.pallas import tpu as pltpu
from jax.experimental.pallas import tpu_sc as plsc

ALPHA = 0.2
EPS = 9e-15
D = 128
CH = 80
LANES = 16
NCORES = 2
NSUB = 16
NPAD = 10240


def _prep_body(x_ref, w_ref, attn_ref, h_ref, s_ref, t_ref):
    h = jnp.dot(x_ref[...], w_ref[...], preferred_element_type=jnp.float32)
    h_ref[...] = h
    a = attn_ref[...].reshape(2, D)
    st = lax.dot_general(h, a, (((1,), (1,)), ((), ())),
                         preferred_element_type=jnp.float32)
    s_ref[...] = st[:, 0]
    t_ref[...] = st[:, 1]


def _combine_body(p_ref, r_ref, o_ref):
    n = o_ref.shape[0]
    p = p_ref[0] + p_ref[1]
    rs = r_ref[0, :n] + r_ref[1, :n]
    o_ref[...] = p / (rs[:, None] + EPS)


def _sc_compiler_params():
    cp = pltpu.CompilerParams()
    if "needs_layout_passes" in pltpu.CompilerParams.__dataclass_fields__:
        cp = dataclasses.replace(cp, needs_layout_passes=False)
    return cp


def _make_scA(n, e):
    CHA = 2000
    esub = e // (NCORES * NSUB)
    csub = esub // CHA
    NBUF = 2

    mesh = plsc.VectorSubcoreMesh(core_axis_name="c", subcore_axis_name="s")

    scratch = [
        pltpu.VMEM((n,), jnp.float32),
        pltpu.VMEM((n,), jnp.float32),
    ]
    scratch += [pltpu.VMEM((CHA,), jnp.int32) for _ in range(2 * NBUF)]
    scratch += [pltpu.VMEM((CHA,), jnp.float32) for _ in range(NBUF)]
    scratch += [pltpu.SemaphoreType.DMA for _ in range(2 * NBUF)]

    @functools.partial(
        pl.kernel,
        out_type=jax.ShapeDtypeStruct((e,), jnp.float32),
        mesh=mesh,
        compiler_params=_sc_compiler_params(),
        scratch_types=scratch,
    )
    def scA(s_hbm, t_hbm, e0_hbm, e1_hbm, w_hbm, s_v, t_v, *bufs):
        e0b = bufs[0:NBUF]
        e1b = bufs[NBUF:2 * NBUF]
        wvb = bufs[2 * NBUF:3 * NBUF]
        isem = bufs[3 * NBUF:4 * NBUF]
        wsem = bufs[4 * NBUF:5 * NBUF]

        core = lax.axis_index("c")
        sid = lax.axis_index("s")

        pltpu.sync_copy(s_hbm, s_v)
        pltpu.sync_copy(t_hbm, t_v)

        def chunk_off(c):
            eb = (core * NSUB + sid) * esub + c * CHA
            return pl.multiple_of(eb, 8)

        def start_idx(c, b):
            eb = chunk_off(c)
            pltpu.make_async_copy(e0_hbm.at[pl.ds(eb, CHA)], e0b[b], isem[b]).start()
            pltpu.make_async_copy(e1_hbm.at[pl.ds(eb, CHA)], e1b[b], isem[b]).start()

        def wait_idx(b):
            pltpu.make_async_copy(e0_hbm.at[pl.ds(0, CHA)], e0b[b], isem[b]).wait()
            pltpu.make_async_copy(e1_hbm.at[pl.ds(0, CHA)], e1b[b], isem[b]).wait()

        def compute_w(c, b):
            @plsc.parallel_loop(0, CHA // LANES, unroll=2)
            def _(g):
                i0 = e0b[b][pl.ds(g * LANES, LANES)]
                i1 = e1b[b][pl.ds(g * LANES, LANES)]
                v = plsc.load_gather(s_v, [i0]) + plsc.load_gather(t_v, [i1])
                v = jnp.where(v > 0, v, ALPHA * v)
                wvb[b][pl.ds(g * LANES, LANES)] = jnp.exp(v)
            eb = chunk_off(c)
            pltpu.make_async_copy(wvb[b], w_hbm.at[pl.ds(eb, CHA)], wsem[b]).start()

        def wait_wb(b):
            pltpu.make_async_copy(wvb[b], w_hbm.at[pl.ds(0, CHA)], wsem[b]).wait()

        for c in range(csub + 1):
            if c <= csub - 1:
                start_idx(c, c % NBUF)
            if c >= 1:
                if c - 3 >= 0:
                    wait_wb((c - 1) % NBUF)
                wait_idx((c - 1) % NBUF)
                compute_w(c - 1, (c - 1) % NBUF)

        for cc in range(csub - NBUF, csub):
            wait_wb(cc % NBUF)

    return scA


def _make_scB(n, e):
    nchunk = e // CH
    chunks_per_core = nchunk // NCORES
    csub = chunks_per_core // NSUB
    zrows = 80
    sub_elems = NPAD // NSUB
    NBUF = 4

    mesh = plsc.VectorSubcoreMesh(core_axis_name="c", subcore_axis_name="s")

    scratch = [
        pltpu.VMEM((sub_elems,), jnp.float32),
        pltpu.VMEM_SHARED((NPAD, D), jnp.float32),
        pltpu.VMEM_SHARED((NPAD,), jnp.float32),
    ]
    scratch += [pltpu.VMEM((CH,), jnp.int32) for _ in range(2 * NBUF)]
    scratch += [pltpu.VMEM((CH, D), jnp.float32) for _ in range(NBUF)]
    scratch += [pltpu.VMEM((CH,), jnp.float32) for _ in range(NBUF)]
    scratch += [pltpu.SemaphoreType.DMA for _ in range(3 * NBUF)]

    @functools.partial(
        pl.kernel,
        out_type=[
            jax.ShapeDtypeStruct((NCORES, n, D), jnp.float32),
            jax.ShapeDtypeStruct((NCORES, NPAD), jnp.float32),
        ],
        mesh=mesh,
        compiler_params=_sc_compiler_params(),
        scratch_types=scratch,
    )
    def scB(h_hbm, w_hbm, e0_hbm, e1_hbm, out_hbm, rs_hbm,
            z1_v, acc_sh, acc1_sh, *bufs):
        e0b = bufs[0:NBUF]
        e1b = bufs[NBUF:2 * NBUF]
        rowsb = bufs[2 * NBUF:3 * NBUF]
        wvb = bufs[3 * NBUF:4 * NBUF]
        isem = bufs[4 * NBUF:5 * NBUF]
        gsem = bufs[5 * NBUF:6 * NBUF]
        ssem = bufs[6 * NBUF:7 * NBUF]

        core = lax.axis_index("c")
        sid = lax.axis_index("s")

        zz = jnp.zeros((LANES,), jnp.float32)

        @pl.loop(0, CH)
        def _(j):
            for k in range(D // LANES):
                rowsb[0][j, pl.ds(k * LANES, LANES)] = zz

        @pl.loop(0, sub_elems // LANES)
        def _(j):
            z1_v[pl.ds(j * LANES, LANES)] = zz

        @pl.loop(sid, NPAD // zrows, step=NSUB)
        def _(t):
            off = pl.multiple_of(t * zrows, 8)
            pltpu.sync_copy(rowsb[0].at[pl.ds(0, zrows)],
                            acc_sh.at[pl.ds(off, zrows)])

        off1 = pl.multiple_of(sid * sub_elems, 128)
        pltpu.sync_copy(z1_v, acc1_sh.at[pl.ds(off1, sub_elems)])
        plsc.subcore_barrier()

        def start_inputs(c, b):
            ci = core * chunks_per_core + sid + NSUB * c
            eb = pl.multiple_of(ci * CH, 8)
            pltpu.make_async_copy(e0_hbm.at[pl.ds(eb, CH)], e0b[b], isem[b]).start()
            pltpu.make_async_copy(e1_hbm.at[pl.ds(eb, CH)], e1b[b], isem[b]).start()
            pltpu.make_async_copy(w_hbm.at[pl.ds(eb, CH)], wvb[b], isem[b]).start()

        def wait_inputs(b):
            pltpu.make_async_copy(e0_hbm.at[pl.ds(0, CH)], e0b[b], isem[b]).wait()
            pltpu.make_async_copy(e1_hbm.at[pl.ds(0, CH)], e1b[b], isem[b]).wait()
            pltpu.make_async_copy(w_hbm.at[pl.ds(0, CH)], wvb[b], isem[b]).wait()

        def start_gather(b):
            pltpu.make_async_copy(h_hbm.at[e1b[b]], rowsb[b], gsem[b]).start()

        def wait_gather(b):
            pltpu.make_async_copy(h_hbm.at[e1b[b]], rowsb[b], gsem[b]).wait()

        def scale(b):
            @plsc.parallel_loop(0, CH, unroll=4)
            def _(j):
                wbr = plsc.load_gather(
                    wvb[b], [j + jnp.zeros((LANES,), jnp.int32)])
                for k in range(D // LANES):
                    sl = pl.ds(k * LANES, LANES)
                    rowsb[b][j, sl] = rowsb[b][j, sl] * wbr

        def start_scatter(b):
            pltpu.make_async_copy(rowsb[b], acc_sh.at[e0b[b]], ssem[b]).start(add=True)
            pltpu.make_async_copy(wvb[b], acc1_sh.at[e0b[b]], ssem[b]).start(add=True)

        def wait_scatter(b):
            pltpu.make_async_copy(rowsb[b], acc_sh.at[e0b[b]], ssem[b]).wait()
            pltpu.make_async_copy(wvb[b], acc1_sh.at[e0b[b]], ssem[b]).wait()

        def iter_(c, b, b1, b2, *, w_scatter, do_idx, do_gather, do_compute):
            if w_scatter:
                wait_scatter(b)
            if do_idx:
                start_inputs(c, b)
            if do_gather:
                wait_inputs(b1)
                start_gather(b1)
            if do_compute:
                wait_gather(b2)
                scale(b2)
                start_scatter(b2)

        for c in range(4):
            iter_(c, c % NBUF, (c - 1) % NBUF, (c - 2) % NBUF,
                  w_scatter=False, do_idx=True, do_gather=(c >= 1),
                  do_compute=(c >= 2))

        @pl.loop(0, (120 - 4) // NBUF)
        def _(m):
            for slot in range(NBUF):
                c = 4 + m * NBUF + slot
                iter_(c, slot, (slot - 1) % NBUF, (slot - 2) % NBUF,
                      w_scatter=True, do_idx=True, do_gather=True,
                      do_compute=True)

        for c in range(120, csub + 2):
            iter_(c, c % NBUF, (c - 1) % NBUF, (c - 2) % NBUF,
                  w_scatter=(c - NBUF <= csub - 1), do_idx=(c <= csub - 1),
                  do_gather=(c - 1 <= csub - 1), do_compute=True)

        wait_scatter((csub - 2) % NBUF)
        wait_scatter((csub - 1) % NBUF)

        plsc.subcore_barrier()

        @pl.loop(sid, n // zrows, step=NSUB)
        def _(t):
            sl = pl.ds(pl.multiple_of(t * zrows, 8), zrows)
            pltpu.sync_copy(acc_sh.at[sl], out_hbm.at[core].at[sl])

        sl1 = pl.ds(off1, sub_elems)
        pltpu.sync_copy(acc1_sh.at[sl1], rs_hbm.at[core].at[sl1])

    return scB


def kernel(x, edge_index, W, attn):
    n = x.shape[0]
    e = edge_index.shape[1]

    h, s, t = pl.pallas_call(
        _prep_body,
        out_shape=[
            jax.ShapeDtypeStruct((n, D), jnp.float32),
            jax.ShapeDtypeStruct((n,), jnp.float32),
            jax.ShapeDtypeStruct((n,), jnp.float32),
        ],
    )(x, W, attn)

    e0 = edge_index[0]
    e1 = edge_index[1]
    w_edge = _make_scA(n, e)(s, t, e0, e1)
    partials, rowsums = _make_scB(n, e)(h, w_edge, e0, e1)

    out = pl.pallas_call(
        _combine_body,
        out_shape=jax.ShapeDtypeStruct((n, D), jnp.float32),
    )(partials, rowsums)
    return out
